# baseline (speedup 1.0000x reference)
import jax
import jax.numpy as jnp
import numpy as np
from jax import lax
from jax.experimental import pallas as pl
from jax.experimental.pallas import tpu as pltpu
from jax.experimental.pallas import tpu_sc as plsc

N_LEVEL = 13
N_ENTRIES = 2 ** 19
MASK = N_ENTRIES - 1
P2 = np.uint32(2654435761).view(np.int32).item()
P3 = np.uint32(805459861).view(np.int32).item()
HIMASK = np.uint32(0xFFFF0000).view(np.int32).item()


def _resolutions():
    b = np.exp((np.log(1024) - np.log(16)) / (N_LEVEL - 1))
    return [int(np.floor(16 * (b ** l))) for l in range(N_LEVEL)]


RES = _resolutions()

N_POINTS = 524288
N_HIDDEN = 64
NC, NS = 2, 16
NW = NC * NS
PW = N_POINTS // NW
C = 1024
NCHUNK = PW // C
VPC = C // 16


def _sc_body(dflat, tpack, enc, xyzidx_v, xyz_v, idx0, idx1, rows0, rows1,
             enc_v, sem0, sem1):
    wid = lax.axis_index("s") * NC + lax.axis_index("c")
    iota3 = lax.iota(jnp.int32, 16) * 3
    idx_bufs = (idx0, idx1)
    rows_bufs = (rows0, rows1)
    sems = (sem0, sem1)
    zero16f = jnp.zeros((16,), jnp.float32)

    for r in range(2 * N_LEVEL, 32):
        def zero_body(i, c2, r=r):
            enc_v[i >> 3, r, pl.ds((i & 7) * 16, 16)] = zero16f
            return c2

        lax.fori_loop(0, (C // 128) * 8, zero_body, 0, unroll=False)

    def chunk_body(k, carry):
        base = wid * PW + k * C

        def xyz_idx_body(i, c2):
            o = i * 16
            xi = 3 * (base + o) + iota3
            xyzidx_v[pl.ds(o, 16)] = xi
            xyzidx_v[pl.ds(C + o, 16)] = xi + 1
            xyzidx_v[pl.ds(2 * C + o, 16)] = xi + 2
            return c2

        lax.fori_loop(0, VPC, xyz_idx_body, 0, unroll=False)
        pltpu.async_copy(dflat.at[xyzidx_v], xyz_v, sem0).wait()

        def idx_pass(l, buf):
            scale = jnp.float32(RES[l] - 1)
            lbase = l * N_ENTRIES

            def idx_body(i, c2):
                o = i * 16
                x = xyz_v[pl.ds(o, 16)] * scale
                y = xyz_v[pl.ds(C + o, 16)] * scale
                z = xyz_v[pl.ds(2 * C + o, 16)] * scale
                x0 = x.astype(jnp.int32)
                y0 = y.astype(jnp.int32)
                z0 = z.astype(jnp.int32)
                ky0 = y0 * P2
                ky1 = ky0 + P2
                kz0 = z0 * P3
                kz1 = kz0 + P3
                for dx in (0, 1):
                    kx = x0 + dx if dx else x0
                    hy0 = kx ^ ky0
                    hy1 = kx ^ ky1
                    for dy in (0, 1):
                        hxy = hy1 if dy else hy0
                        for dz in (0, 1):
                            kz = kz1 if dz else kz0
                            h = ((hxy ^ kz) & MASK) + lbase
                            cidx = dx * 4 + dy * 2 + dz
                            buf[pl.ds(cidx * C + o, 16)] = h
                return c2

            lax.fori_loop(0, VPC, idx_body, 0, unroll=False)

        def acc_pass(l, rbuf):
            scale = jnp.float32(RES[l] - 1)

            def acc_body(i, c2):
                o = i * 16
                x = xyz_v[pl.ds(o, 16)] * scale
                y = xyz_v[pl.ds(C + o, 16)] * scale
                z = xyz_v[pl.ds(2 * C + o, 16)] * scale
                fx = x - x.astype(jnp.int32).astype(jnp.float32)
                fy = y - y.astype(jnp.int32).astype(jnp.float32)
                fz = z - z.astype(jnp.int32).astype(jnp.float32)
                wx = (1.0 - fx, fx)
                wy = (1.0 - fy, fy)
                wz = (1.0 - fz, fz)
                acc0 = jnp.zeros((16,), jnp.float32)
                acc1 = jnp.zeros((16,), jnp.float32)
                for dx in (0, 1):
                    for dy in (0, 1):
                        wxy = wx[dx] * wy[dy]
                        for dz in (0, 1):
                            cidx = dx * 4 + dy * 2 + dz
                            val = rbuf[pl.ds(cidx * C + o, 16)]
                            f0 = plsc.bitcast(val << 16, jnp.float32)
                            f1 = plsc.bitcast(val & HIMASK, jnp.float32)
                            w = wxy * wz[dz]
                            acc0 = acc0 + w * f0
                            acc1 = acc1 + w * f1
                b = o >> 7
                cpos = o & 127
                enc_v[b, 2 * l, pl.ds(cpos, 16)] = acc0
                enc_v[b, 2 * l + 1, pl.ds(cpos, 16)] = acc1
                return c2

            lax.fori_loop(0, VPC, acc_body, 0, unroll=False)

        idx_pass(0, idx_bufs[0])
        dmas = [pltpu.async_copy(tpack.at[idx_bufs[0]], rows_bufs[0], sems[0])]
        for l in range(1, N_LEVEL):
            p = l % 2
            idx_pass(l, idx_bufs[p])
            dmas.append(
                pltpu.async_copy(tpack.at[idx_bufs[p]], rows_bufs[p], sems[p]))
            dmas[l - 1].wait()
            acc_pass(l - 1, rows_bufs[(l - 1) % 2])
        dmas[N_LEVEL - 1].wait()
        acc_pass(N_LEVEL - 1, rows_bufs[(N_LEVEL - 1) % 2])
        pltpu.sync_copy(enc_v, enc.at[pl.ds(wid * (PW // 128) + k * (C // 128),
                                            C // 128)])
        return carry

    lax.fori_loop(0, NCHUNK, chunk_body, 0, unroll=False)


TW = N_LEVEL * N_ENTRIES // NW
CP = 4096
NPCHUNK = TW // CP


def _sc_pack_body(tf32, tpk, fb0, fb1, ob, psem0, psem1):
    wid = lax.axis_index("s") * NC + lax.axis_index("c")
    iota2 = lax.iota(jnp.int32, 16) * 2
    fbufs = (fb0, fb1)
    sems = (psem0, psem1)
    base = wid * TW

    total_w = 2 * N_LEVEL * N_ENTRIES

    pltpu.async_copy(tf32.at[pl.ds(2 * base, 2 * CP)], fbufs[0], sems[0])

    def pack_chunk(k, carry):
        for p in (0, 1):
            kk = 2 * k + p
            off_next = (2 * (base + (kk + 1) * CP)) % total_w
            pltpu.async_copy(tf32.at[pl.ds(off_next, 2 * CP)], fbufs[1 - p],
                             sems[1 - p])
            pltpu.make_async_copy(tf32.at[pl.ds(0, 2 * CP)], fbufs[p],
                                  sems[p]).wait()

            def pack_body(j, c2, p=p):
                o = j * 16
                ii = 2 * o + iota2
                f0 = plsc.load_gather(fbufs[p], [ii])
                f1 = plsc.load_gather(fbufs[p], [ii + 1])
                ob[pl.ds(o, 16)] = plsc.bitcast(
                    plsc.pack(f0, f1, format=plsc.PackFormat.INTERLEAVED),
                    jnp.int32)
                return c2

            lax.fori_loop(0, CP // 16, pack_body, 0, unroll=False)
            pltpu.sync_copy(ob, tpk.at[pl.ds(base + kk * CP, CP)])
        return carry

    lax.fori_loop(0, NPCHUNK // 2, pack_chunk, 0, unroll=False)
    pltpu.make_async_copy(tf32.at[pl.ds(0, 2 * CP)], fbufs[0], sems[0]).wait()


@jax.jit
def _sc_pack(tf32):
    mesh = plsc.VectorSubcoreMesh(
        core_axis_name="c", subcore_axis_name="s", num_cores=NC, num_subcores=NS
    )
    return pl.kernel(
        _sc_pack_body,
        out_type=jax.ShapeDtypeStruct((N_LEVEL * N_ENTRIES,), jnp.int32),
        mesh=mesh,
        scratch_types=[
            pltpu.VMEM((2 * CP,), jnp.float32),
            pltpu.VMEM((2 * CP,), jnp.float32),
            pltpu.VMEM((CP,), jnp.int32),
            pltpu.SemaphoreType.DMA,
            pltpu.SemaphoreType.DMA,
        ],
        compiler_params=pltpu.CompilerParams(
            use_tc_tiling_on_sc=False, needs_layout_passes=False),
    )(tf32)


@jax.jit
def _sc_encode(dflat, tpack):
    mesh = plsc.VectorSubcoreMesh(
        core_axis_name="c", subcore_axis_name="s", num_cores=NC, num_subcores=NS
    )
    return pl.kernel(
        _sc_body,
        out_type=jax.ShapeDtypeStruct((N_POINTS // 128, 32, 128), jnp.float32),
        mesh=mesh,
        scratch_types=[
            pltpu.VMEM((3 * C,), jnp.int32),
            pltpu.VMEM((3 * C,), jnp.float32),
            pltpu.VMEM((8 * C,), jnp.int32),
            pltpu.VMEM((8 * C,), jnp.int32),
            pltpu.VMEM((8 * C,), jnp.int32),
            pltpu.VMEM((8 * C,), jnp.int32),
            pltpu.VMEM((C // 128, 32, 128), jnp.float32),
            pltpu.SemaphoreType.DMA,
            pltpu.SemaphoreType.DMA,
        ],
        compiler_params=pltpu.CompilerParams(
            use_tc_tiling_on_sc=False, needs_layout_passes=False),
    )(dflat, tpack)


NB = 8192
NB128 = NB // 128


def _mlp_body(enc_b, w1tp, b1, w2t, b2, w3t, b3, out_b):
    prec = jax.lax.Precision.HIGHEST
    x = jnp.concatenate([enc_b[j] for j in range(NB128)], axis=1)
    h = jnp.dot(w1tp[...], x, preferred_element_type=jnp.float32,
                precision=prec) + b1[...]
    h = jax.nn.gelu(h)
    h = jnp.dot(w2t[...], h, preferred_element_type=jnp.float32,
                precision=prec) + b2[...]
    h = jax.nn.gelu(h)
    y = jnp.dot(w3t[...], h, preferred_element_type=jnp.float32,
                precision=prec) + b3[...]
    for j in range(NB128):
        out_b[j] = y[:, j * 128:(j + 1) * 128]


@jax.jit
def _mlp(enc, w1tp, b1, w2t, b2, w3t, b3):
    nh = w1tp.shape[0]
    grid = (N_POINTS // NB,)
    return pl.pallas_call(
        _mlp_body,
        grid=grid,
        in_specs=[
            pl.BlockSpec((NB128, 32, 128), lambda i: (i, 0, 0)),
            pl.BlockSpec((nh, 32), lambda i: (0, 0)),
            pl.BlockSpec((nh, 1), lambda i: (0, 0)),
            pl.BlockSpec((nh, nh), lambda i: (0, 0)),
            pl.BlockSpec((nh, 1), lambda i: (0, 0)),
            pl.BlockSpec((1, nh), lambda i: (0, 0)),
            pl.BlockSpec((1, 1), lambda i: (0, 0)),
        ],
        out_specs=pl.BlockSpec((NB128, 1, 128), lambda i: (i, 0, 0)),
        out_shape=jax.ShapeDtypeStruct((N_POINTS // 128, 1, 128), jnp.float32),
    )(enc, w1tp, b1, w2t, b2, w3t, b3)


def kernel(data, tables, W1, b1, W2, b2, W3, b3):
    dflat = data.reshape(3 * N_POINTS)
    tpack = _sc_pack(tables.reshape(2 * N_LEVEL * N_ENTRIES))
    enc = _sc_encode(dflat, tpack)
    w1tp = jnp.zeros((N_HIDDEN, 32), jnp.float32).at[:, :2 * N_LEVEL].set(W1.T)
    out = _mlp(enc, w1tp, b1[:, None], W2.T, b2[:, None], W3.T, b3[:, None])
    return out.reshape(N_POINTS, 1)


# speedup vs baseline: 3.6185x; 3.6185x over previous
import jax
import jax.numpy as jnp
import numpy as np
from jax import lax
from jax.experimental import pallas as pl
from jax.experimental.pallas import tpu as pltpu
from jax.experimental.pallas import tpu_sc as plsc

N_LEVEL = 13
N_ENTRIES = 2 ** 19
MASK = N_ENTRIES - 1
P2 = np.uint32(2654435761).view(np.int32).item()
P3 = np.uint32(805459861).view(np.int32).item()
HIMASK = np.uint32(0xFFFF0000).view(np.int32).item()


def _resolutions():
    b = np.exp((np.log(1024) - np.log(16)) / (N_LEVEL - 1))
    return [int(np.floor(16 * (b ** l))) for l in range(N_LEVEL)]


RES = _resolutions()

N_POINTS = 524288
N_HIDDEN = 64
NC, NS = 2, 16
NW = NC * NS
PW = N_POINTS // NW
C = 1024
NCHUNK = PW // C
VPC = C // 16


def _sc_body(dflat, tpack, enc, xyzidx_v, xyz_v, idx0, idx1, rows0, rows1,
             enc_v, sem0, sem1):
    wid = lax.axis_index("s") * NC + lax.axis_index("c")
    iota3 = lax.iota(jnp.int32, 16) * 3
    idx_bufs = (idx0, idx1)
    rows_bufs = (rows0, rows1)
    sems = (sem0, sem1)
    zero16f = jnp.zeros((16,), jnp.float32)

    for r in range(2 * N_LEVEL, 32):
        def zero_body(i, c2, r=r):
            enc_v[i >> 3, r, pl.ds((i & 7) * 16, 16)] = zero16f
            return c2

        lax.fori_loop(0, (C // 128) * 8, zero_body, 0, unroll=False)

    def chunk_body(k, carry):
        base = wid * PW + k * C

        def xyz_idx_body(i, c2):
            o = i * 16
            xi = 3 * (base + o) + iota3
            xyzidx_v[pl.ds(o, 16)] = xi
            xyzidx_v[pl.ds(C + o, 16)] = xi + 1
            xyzidx_v[pl.ds(2 * C + o, 16)] = xi + 2
            return c2

        lax.fori_loop(0, VPC, xyz_idx_body, 0, unroll=False)
        pltpu.async_copy(dflat.at[xyzidx_v], xyz_v, sem0).wait()

        def idx_pass(l, buf):
            scale = jnp.float32(RES[l] - 1)
            lbase = l * N_ENTRIES

            def idx_body(i, c2):
                o = i * 16
                x = xyz_v[pl.ds(o, 16)] * scale
                y = xyz_v[pl.ds(C + o, 16)] * scale
                z = xyz_v[pl.ds(2 * C + o, 16)] * scale
                x0 = x.astype(jnp.int32)
                y0 = y.astype(jnp.int32)
                z0 = z.astype(jnp.int32)
                ky0 = y0 * P2
                ky1 = ky0 + P2
                kz0 = z0 * P3
                kz1 = kz0 + P3
                for dx in (0, 1):
                    kx = x0 + dx if dx else x0
                    hy0 = kx ^ ky0
                    hy1 = kx ^ ky1
                    for dy in (0, 1):
                        hxy = hy1 if dy else hy0
                        for dz in (0, 1):
                            kz = kz1 if dz else kz0
                            h = ((hxy ^ kz) & MASK) + lbase
                            cidx = dx * 4 + dy * 2 + dz
                            buf[pl.ds(cidx * C + o, 16)] = h
                return c2

            lax.fori_loop(0, VPC, idx_body, 0, unroll=False)

        def acc_pass(l, rbuf):
            scale = jnp.float32(RES[l] - 1)

            def acc_body(i, c2):
                o = i * 16
                x = xyz_v[pl.ds(o, 16)] * scale
                y = xyz_v[pl.ds(C + o, 16)] * scale
                z = xyz_v[pl.ds(2 * C + o, 16)] * scale
                fx = x - x.astype(jnp.int32).astype(jnp.float32)
                fy = y - y.astype(jnp.int32).astype(jnp.float32)
                fz = z - z.astype(jnp.int32).astype(jnp.float32)
                wx = (1.0 - fx, fx)
                wy = (1.0 - fy, fy)
                wz = (1.0 - fz, fz)
                acc0 = jnp.zeros((16,), jnp.float32)
                acc1 = jnp.zeros((16,), jnp.float32)
                for dx in (0, 1):
                    for dy in (0, 1):
                        wxy = wx[dx] * wy[dy]
                        for dz in (0, 1):
                            cidx = dx * 4 + dy * 2 + dz
                            val = rbuf[pl.ds(cidx * C + o, 16)]
                            f0 = plsc.bitcast(val << 16, jnp.float32)
                            f1 = plsc.bitcast(val & HIMASK, jnp.float32)
                            w = wxy * wz[dz]
                            acc0 = acc0 + w * f0
                            acc1 = acc1 + w * f1
                b = o >> 7
                cpos = o & 127
                enc_v[b, 2 * l, pl.ds(cpos, 16)] = acc0
                enc_v[b, 2 * l + 1, pl.ds(cpos, 16)] = acc1
                return c2

            lax.fori_loop(0, VPC, acc_body, 0, unroll=False)

        idx_pass(0, idx_bufs[0])
        dmas = [pltpu.async_copy(tpack.at[idx_bufs[0]], rows_bufs[0], sems[0])]
        for l in range(1, N_LEVEL):
            p = l % 2
            idx_pass(l, idx_bufs[p])
            dmas.append(
                pltpu.async_copy(tpack.at[idx_bufs[p]], rows_bufs[p], sems[p]))
            dmas[l - 1].wait()
            acc_pass(l - 1, rows_bufs[(l - 1) % 2])
        dmas[N_LEVEL - 1].wait()
        acc_pass(N_LEVEL - 1, rows_bufs[(N_LEVEL - 1) % 2])
        pltpu.sync_copy(enc_v, enc.at[pl.ds(wid * (PW // 128) + k * (C // 128),
                                            C // 128)])
        return carry

    lax.fori_loop(0, NCHUNK, chunk_body, 0, unroll=False)


@jax.jit
def _sc_encode(dflat, tpack):
    mesh = plsc.VectorSubcoreMesh(
        core_axis_name="c", subcore_axis_name="s", num_cores=NC, num_subcores=NS
    )
    return pl.kernel(
        _sc_body,
        out_type=jax.ShapeDtypeStruct((N_POINTS // 128, 32, 128), jnp.float32),
        mesh=mesh,
        scratch_types=[
            pltpu.VMEM((3 * C,), jnp.int32),
            pltpu.VMEM((3 * C,), jnp.float32),
            pltpu.VMEM((8 * C,), jnp.int32),
            pltpu.VMEM((8 * C,), jnp.int32),
            pltpu.VMEM((8 * C,), jnp.int32),
            pltpu.VMEM((8 * C,), jnp.int32),
            pltpu.VMEM((C // 128, 32, 128), jnp.float32),
            pltpu.SemaphoreType.DMA,
            pltpu.SemaphoreType.DMA,
        ],
        compiler_params=pltpu.CompilerParams(
            use_tc_tiling_on_sc=False, needs_layout_passes=False),
    )(dflat, tpack)


NB = 8192
NB128 = NB // 128


def _mlp_body(enc_b, w1tp, b1, w2t, b2, w3t, b3, out_b):
    prec = jax.lax.Precision.DEFAULT
    x = jnp.concatenate([enc_b[j] for j in range(NB128)], axis=1)
    h = jnp.dot(w1tp[...], x, preferred_element_type=jnp.float32,
                precision=prec) + b1[...]
    h = jax.nn.gelu(h)
    h = jnp.dot(w2t[...], h, preferred_element_type=jnp.float32,
                precision=prec) + b2[...]
    h = jax.nn.gelu(h)
    y = jnp.dot(w3t[...], h, preferred_element_type=jnp.float32,
                precision=prec) + b3[...]
    for j in range(NB128):
        out_b[j] = y[:, j * 128:(j + 1) * 128]


@jax.jit
def _mlp(enc, w1tp, b1, w2t, b2, w3t, b3):
    nh = w1tp.shape[0]
    grid = (N_POINTS // NB,)
    return pl.pallas_call(
        _mlp_body,
        grid=grid,
        in_specs=[
            pl.BlockSpec((NB128, 32, 128), lambda i: (i, 0, 0)),
            pl.BlockSpec((nh, 32), lambda i: (0, 0)),
            pl.BlockSpec((nh, 1), lambda i: (0, 0)),
            pl.BlockSpec((nh, nh), lambda i: (0, 0)),
            pl.BlockSpec((nh, 1), lambda i: (0, 0)),
            pl.BlockSpec((1, nh), lambda i: (0, 0)),
            pl.BlockSpec((1, 1), lambda i: (0, 0)),
        ],
        out_specs=pl.BlockSpec((NB128, 1, 128), lambda i: (i, 0, 0)),
        out_shape=jax.ShapeDtypeStruct((N_POINTS // 128, 1, 128), jnp.float32),
    )(enc, w1tp, b1, w2t, b2, w3t, b3)


def kernel(data, tables, W1, b1, W2, b2, W3, b3):
    dflat = data.reshape(3 * N_POINTS)
    tpack = jax.lax.bitcast_convert_type(
        tables.astype(jnp.bfloat16), jnp.int32).reshape(N_LEVEL * N_ENTRIES)
    enc = _sc_encode(dflat, tpack)
    w1tp = jnp.zeros((N_HIDDEN, 32), jnp.float32).at[:, :2 * N_LEVEL].set(W1.T)
    out = _mlp(enc, w1tp, b1[:, None], W2.T, b2[:, None], W3.T, b3[:, None])
    return out.reshape(N_POINTS, 1)


# speedup vs baseline: 3.6359x; 1.0048x over previous
import jax
import jax.numpy as jnp
import numpy as np
from jax import lax
from jax.experimental import pallas as pl
from jax.experimental.pallas import tpu as pltpu
from jax.experimental.pallas import tpu_sc as plsc

N_LEVEL = 13
N_ENTRIES = 2 ** 19
MASK = N_ENTRIES - 1
P2 = np.uint32(2654435761).view(np.int32).item()
P3 = np.uint32(805459861).view(np.int32).item()
HIMASK = np.uint32(0xFFFF0000).view(np.int32).item()


def _resolutions():
    b = np.exp((np.log(1024) - np.log(16)) / (N_LEVEL - 1))
    return [int(np.floor(16 * (b ** l))) for l in range(N_LEVEL)]


RES = _resolutions()

N_POINTS = 524288
N_HIDDEN = 64
NC, NS = 2, 16
NW = NC * NS
PW = N_POINTS // NW
C = 1024
NCHUNK = PW // C
VPC = C // 16


NBUF = 3


def _sc_body(dflat, tpack, enc, xyzidx_v, xyz_v, idx0, idx1, idx2,
             rows0, rows1, rows2, enc_v, sem0, sem1, sem2):
    wid = lax.axis_index("s") * NC + lax.axis_index("c")
    iota3 = lax.iota(jnp.int32, 16) * 3
    idx_bufs = (idx0, idx1, idx2)
    rows_bufs = (rows0, rows1, rows2)
    sems = (sem0, sem1, sem2)
    zero16f = jnp.zeros((16,), jnp.float32)

    for r in range(2 * N_LEVEL, 32):
        def zero_body(i, c2, r=r):
            enc_v[i >> 3, r, pl.ds((i & 7) * 16, 16)] = zero16f
            return c2

        lax.fori_loop(0, (C // 128) * 8, zero_body, 0, unroll=False)

    def chunk_body(k, carry):
        base = wid * PW + k * C

        def xyz_idx_body(i, c2):
            o = i * 16
            xi = 3 * (base + o) + iota3
            xyzidx_v[pl.ds(o, 16)] = xi
            xyzidx_v[pl.ds(C + o, 16)] = xi + 1
            xyzidx_v[pl.ds(2 * C + o, 16)] = xi + 2
            return c2

        lax.fori_loop(0, VPC, xyz_idx_body, 0, unroll=False)
        pltpu.async_copy(dflat.at[xyzidx_v], xyz_v, sem0).wait()

        def idx_pass(l, buf):
            scale = jnp.float32(RES[l] - 1)
            lbase = l * N_ENTRIES

            def idx_body(i, c2):
                o = i * 16
                x = xyz_v[pl.ds(o, 16)] * scale
                y = xyz_v[pl.ds(C + o, 16)] * scale
                z = xyz_v[pl.ds(2 * C + o, 16)] * scale
                x0 = x.astype(jnp.int32)
                y0 = y.astype(jnp.int32)
                z0 = z.astype(jnp.int32)
                ky0 = y0 * P2
                ky1 = ky0 + P2
                kz0 = z0 * P3
                kz1 = kz0 + P3
                for dx in (0, 1):
                    kx = x0 + dx if dx else x0
                    hy0 = kx ^ ky0
                    hy1 = kx ^ ky1
                    for dy in (0, 1):
                        hxy = hy1 if dy else hy0
                        for dz in (0, 1):
                            kz = kz1 if dz else kz0
                            h = ((hxy ^ kz) & MASK) + lbase
                            cidx = dx * 4 + dy * 2 + dz
                            buf[pl.ds(cidx * C + o, 16)] = h
                return c2

            lax.fori_loop(0, VPC, idx_body, 0, unroll=False)

        def acc_pass(l, rbuf):
            scale = jnp.float32(RES[l] - 1)

            def acc_body(i, c2):
                o = i * 16
                x = xyz_v[pl.ds(o, 16)] * scale
                y = xyz_v[pl.ds(C + o, 16)] * scale
                z = xyz_v[pl.ds(2 * C + o, 16)] * scale
                fx = x - x.astype(jnp.int32).astype(jnp.float32)
                fy = y - y.astype(jnp.int32).astype(jnp.float32)
                fz = z - z.astype(jnp.int32).astype(jnp.float32)
                wx = (1.0 - fx, fx)
                wy = (1.0 - fy, fy)
                wz = (1.0 - fz, fz)
                acc0 = jnp.zeros((16,), jnp.float32)
                acc1 = jnp.zeros((16,), jnp.float32)
                for dx in (0, 1):
                    for dy in (0, 1):
                        wxy = wx[dx] * wy[dy]
                        for dz in (0, 1):
                            cidx = dx * 4 + dy * 2 + dz
                            val = rbuf[pl.ds(cidx * C + o, 16)]
                            f0 = plsc.bitcast(val << 16, jnp.float32)
                            f1 = plsc.bitcast(val & HIMASK, jnp.float32)
                            w = wxy * wz[dz]
                            acc0 = acc0 + w * f0
                            acc1 = acc1 + w * f1
                b = o >> 7
                cpos = o & 127
                enc_v[b, 2 * l, pl.ds(cpos, 16)] = acc0
                enc_v[b, 2 * l + 1, pl.ds(cpos, 16)] = acc1
                return c2

            lax.fori_loop(0, VPC, acc_body, 0, unroll=False)

        dmas = []
        for l in range(NBUF - 1):
            idx_pass(l, idx_bufs[l])
            dmas.append(
                pltpu.async_copy(tpack.at[idx_bufs[l]], rows_bufs[l], sems[l]))
        for l in range(NBUF - 1, N_LEVEL):
            p = l % NBUF
            idx_pass(l, idx_bufs[p])
            dmas.append(
                pltpu.async_copy(tpack.at[idx_bufs[p]], rows_bufs[p], sems[p]))
            q = l - (NBUF - 1)
            dmas[q].wait()
            acc_pass(q, rows_bufs[q % NBUF])
        for l in range(N_LEVEL - (NBUF - 1), N_LEVEL):
            dmas[l].wait()
            acc_pass(l, rows_bufs[l % NBUF])
        pltpu.sync_copy(enc_v, enc.at[pl.ds(wid * (PW // 128) + k * (C // 128),
                                            C // 128)])
        return carry

    lax.fori_loop(0, NCHUNK, chunk_body, 0, unroll=False)


@jax.jit
def _sc_encode(dflat, tpack):
    mesh = plsc.VectorSubcoreMesh(
        core_axis_name="c", subcore_axis_name="s", num_cores=NC, num_subcores=NS
    )
    return pl.kernel(
        _sc_body,
        out_type=jax.ShapeDtypeStruct((N_POINTS // 128, 32, 128), jnp.float32),
        mesh=mesh,
        scratch_types=[
            pltpu.VMEM((3 * C,), jnp.int32),
            pltpu.VMEM((3 * C,), jnp.float32),
            pltpu.VMEM((8 * C,), jnp.int32),
            pltpu.VMEM((8 * C,), jnp.int32),
            pltpu.VMEM((8 * C,), jnp.int32),
            pltpu.VMEM((8 * C,), jnp.int32),
            pltpu.VMEM((8 * C,), jnp.int32),
            pltpu.VMEM((8 * C,), jnp.int32),
            pltpu.VMEM((C // 128, 32, 128), jnp.float32),
            pltpu.SemaphoreType.DMA,
            pltpu.SemaphoreType.DMA,
            pltpu.SemaphoreType.DMA,
        ],
        compiler_params=pltpu.CompilerParams(
            use_tc_tiling_on_sc=False, needs_layout_passes=False),
    )(dflat, tpack)


NB = 8192
NB128 = NB // 128


def _mlp_body(enc_b, w1tp, b1, w2t, b2, w3t, b3, out_b):
    prec = jax.lax.Precision.DEFAULT
    x = jnp.concatenate([enc_b[j] for j in range(NB128)], axis=1)
    h = jnp.dot(w1tp[...], x, preferred_element_type=jnp.float32,
                precision=prec) + b1[...]
    h = jax.nn.gelu(h)
    h = jnp.dot(w2t[...], h, preferred_element_type=jnp.float32,
                precision=prec) + b2[...]
    h = jax.nn.gelu(h)
    y = jnp.dot(w3t[...], h, preferred_element_type=jnp.float32,
                precision=prec) + b3[...]
    for j in range(NB128):
        out_b[j] = y[:, j * 128:(j + 1) * 128]


@jax.jit
def _mlp(enc, w1tp, b1, w2t, b2, w3t, b3):
    nh = w1tp.shape[0]
    grid = (N_POINTS // NB,)
    return pl.pallas_call(
        _mlp_body,
        grid=grid,
        in_specs=[
            pl.BlockSpec((NB128, 32, 128), lambda i: (i, 0, 0)),
            pl.BlockSpec((nh, 32), lambda i: (0, 0)),
            pl.BlockSpec((nh, 1), lambda i: (0, 0)),
            pl.BlockSpec((nh, nh), lambda i: (0, 0)),
            pl.BlockSpec((nh, 1), lambda i: (0, 0)),
            pl.BlockSpec((1, nh), lambda i: (0, 0)),
            pl.BlockSpec((1, 1), lambda i: (0, 0)),
        ],
        out_specs=pl.BlockSpec((NB128, 1, 128), lambda i: (i, 0, 0)),
        out_shape=jax.ShapeDtypeStruct((N_POINTS // 128, 1, 128), jnp.float32),
    )(enc, w1tp, b1, w2t, b2, w3t, b3)


def kernel(data, tables, W1, b1, W2, b2, W3, b3):
    dflat = data.reshape(3 * N_POINTS)
    tpack = jax.lax.bitcast_convert_type(
        tables.astype(jnp.bfloat16), jnp.int32).reshape(N_LEVEL * N_ENTRIES)
    enc = _sc_encode(dflat, tpack)
    w1tp = jnp.zeros((N_HIDDEN, 32), jnp.float32).at[:, :2 * N_LEVEL].set(W1.T)
    out = _mlp(enc, w1tp, b1[:, None], W2.T, b2[:, None], W3.T, b3[:, None])
    return out.reshape(N_POINTS, 1)
